# baseline (speedup 1.0000x reference)
import functools

import jax
import jax.numpy as jnp
from jax import lax
from jax.experimental import pallas as pl
from jax.experimental.pallas import tpu as pltpu
from jax.experimental.pallas import tpu_sc as plsc

B, S, D = 4, 8192, 2048
H = D // 2
E = 64

S_TC = 6144
S_SC = S - S_TC
S_BLK = 512
COLS = 256
S_CHUNK = 128
N_CHUNKS = S_SC // S_CHUNK
NV = COLS // 16


def _tc_pool_kernel(x_ref, out_ref, acc_ref):
    i = pl.program_id(0)
    n = pl.num_programs(0)
    part = jnp.sum(x_ref[...], axis=1)

    @pl.when(i == 0)
    def _init():
        acc_ref[...] = part

    @pl.when(i > 0)
    def _accum():
        acc_ref[...] += part

    @pl.when(i == n - 1)
    def _out():
        out_ref[...] = acc_ref[...]


def _combine_kernel(pa_ref, pb_ref, w1_ref, b1_ref, w2_ref, b2_ref, out_ref):
    pooled = (pa_ref[...] + pb_ref[...]) * (1.0 / S)
    h = lax.dot_general(
        pooled, w1_ref[...], (((1,), (0,)), ((), ())),
        preferred_element_type=jnp.float32,
    ) + b1_ref[...]
    h = 0.5 * h * (1.0 + lax.erf(h * (2.0 ** -0.5)))
    logits = lax.dot_general(
        h, w2_ref[...], (((1,), (0,)), ((), ())),
        preferred_element_type=jnp.float32,
    ) + b2_ref[...]
    m = jnp.max(logits, axis=-1, keepdims=True)
    e = jnp.exp(logits - m)
    out_ref[...] = e / jnp.sum(e, axis=-1, keepdims=True)


def _sc_pool_body(x_hbm, out_hbm, buf0, buf1, stage, sem0, sem1):
    nc = 2
    wid = lax.axis_index("s") * nc + lax.axis_index("c")
    b = wid // 8
    c0 = (wid % 8) * COLS

    def start(i, buf, sem):
        s0 = S_TC + i * S_CHUNK
        return pltpu.async_copy(
            x_hbm.at[b, pl.ds(s0, S_CHUNK), pl.ds(c0, COLS)], buf, sem)

    start(0, buf0, sem0)

    def accum(buf, accs):
        def row(s, a):
            return tuple(a[k] + buf[s, pl.ds(k * 16, 16)] for k in range(NV))
        return lax.fori_loop(0, S_CHUNK, row, accs)

    def chunk_pair(i, accs):
        start(2 * i + 1, buf1, sem1)
        pltpu.make_async_copy(
            x_hbm.at[b, pl.ds(S_TC, S_CHUNK), pl.ds(c0, COLS)], buf0, sem0
        ).wait()
        accs = accum(buf0, accs)

        @pl.when(2 * i + 2 < N_CHUNKS)
        def _():
            start(2 * i + 2, buf0, sem0)

        pltpu.make_async_copy(
            x_hbm.at[b, pl.ds(S_TC, S_CHUNK), pl.ds(c0, COLS)], buf1, sem1
        ).wait()
        accs = accum(buf1, accs)
        return accs

    assert N_CHUNKS % 2 == 0
    accs = tuple(jnp.zeros((16,), jnp.float32) for _ in range(NV))
    accs = lax.fori_loop(0, N_CHUNKS // 2, chunk_pair, accs)

    for k in range(NV):
        stage[pl.ds(k * 16, 16)] = accs[k]
    pltpu.sync_copy(stage, out_hbm.at[b, pl.ds(c0, COLS)])


def _sc_pool(x):
    mesh = plsc.VectorSubcoreMesh(core_axis_name="c", subcore_axis_name="s")
    f = functools.partial(
        pl.kernel,
        mesh=mesh,
        out_type=jax.ShapeDtypeStruct((B, D), jnp.float32),
        scratch_types=[
            pltpu.VMEM((S_CHUNK, COLS), jnp.float32),
            pltpu.VMEM((S_CHUNK, COLS), jnp.float32),
            pltpu.VMEM((COLS,), jnp.float32),
            pltpu.SemaphoreType.DMA,
            pltpu.SemaphoreType.DMA,
        ],
    )(_sc_pool_body)
    return f(x)


def _tc_pool(x):
    n_blk = S_TC // S_BLK
    return pl.pallas_call(
        _tc_pool_kernel,
        grid=(n_blk,),
        in_specs=[pl.BlockSpec((B, S_BLK, D), lambda i: (0, i, 0))],
        out_specs=pl.BlockSpec((B, D), lambda i: (0, 0)),
        out_shape=jax.ShapeDtypeStruct((B, D), jnp.float32),
        scratch_shapes=[pltpu.VMEM((B, D), jnp.float32)],
        compiler_params=pltpu.CompilerParams(
            dimension_semantics=("arbitrary",),
        ),
    )(x)


def _combine(pa, pb, W1, b1, W2, b2):
    return pl.pallas_call(
        _combine_kernel,
        out_shape=jax.ShapeDtypeStruct((B, E), jnp.float32),
    )(pa, pb, W1, b1.reshape(1, H), W2, b2.reshape(1, E))


@jax.jit
def kernel(x, W1, b1, W2, b2):
    pooled_sc = _sc_pool(x)
    pooled_tc = _tc_pool(x)
    return _combine(pooled_tc, pooled_sc, W1, b1, W2, b2)


# speedup vs baseline: 1.2229x; 1.2229x over previous
import jax
import jax.numpy as jnp
from jax import lax
from jax.experimental import pallas as pl
from jax.experimental.pallas import tpu as pltpu

B, S, D = 4, 8192, 2048
H = D // 2
E = 64
S_BLK = 1024


def _router_kernel(x_ref, w1_ref, b1_ref, w2_ref, b2_ref, out_ref, acc_ref):
    b = pl.program_id(0)
    j = pl.program_id(1)
    nj = pl.num_programs(1)

    part = jnp.sum(x_ref[0], axis=0, keepdims=True)

    @pl.when(j == 0)
    def _init():
        acc_ref[pl.ds(b, 1), :] = part

    @pl.when(j > 0)
    def _accum():
        acc_ref[pl.ds(b, 1), :] += part

    @pl.when((b == B - 1) & (j == nj - 1))
    def _finish():
        pooled = acc_ref[...] * (1.0 / S)
        h = lax.dot_general(
            pooled, w1_ref[...], (((1,), (0,)), ((), ())),
            preferred_element_type=jnp.float32,
        ) + b1_ref[...]
        h = 0.5 * h * (1.0 + lax.erf(h * (2.0 ** -0.5)))
        logits = lax.dot_general(
            h, w2_ref[...], (((1,), (0,)), ((), ())),
            preferred_element_type=jnp.float32,
        ) + b2_ref[...]
        m = jnp.max(logits, axis=-1, keepdims=True)
        e = jnp.exp(logits - m)
        out_ref[...] = e / jnp.sum(e, axis=-1, keepdims=True)


@jax.jit
def kernel(x, W1, b1, W2, b2):
    grid = (B, S // S_BLK)
    out = pl.pallas_call(
        _router_kernel,
        grid=grid,
        in_specs=[
            pl.BlockSpec((1, S_BLK, D), lambda b, j: (b, j, 0)),
            pl.BlockSpec((D, H), lambda b, j: (0, 0)),
            pl.BlockSpec((1, H), lambda b, j: (0, 0)),
            pl.BlockSpec((H, E), lambda b, j: (0, 0)),
            pl.BlockSpec((1, E), lambda b, j: (0, 0)),
        ],
        out_specs=pl.BlockSpec((B, E), lambda b, j: (0, 0)),
        out_shape=jax.ShapeDtypeStruct((B, E), jnp.float32),
        scratch_shapes=[pltpu.VMEM((B, D), jnp.float32)],
        compiler_params=pltpu.CompilerParams(
            dimension_semantics=("arbitrary", "arbitrary"),
        ),
    )(x, W1, b1.reshape(1, H), W2, b2.reshape(1, E))
    return out


# speedup vs baseline: 1.2234x; 1.0004x over previous
import jax
import jax.numpy as jnp
from jax import lax
from jax.experimental import pallas as pl
from jax.experimental.pallas import tpu as pltpu

B, S, D = 4, 8192, 2048
H = D // 2
E = 64
S_BLK = 2048


def _router_kernel(x_ref, w1_ref, b1_ref, w2_ref, b2_ref, out_ref, acc_ref):
    b = pl.program_id(0)
    j = pl.program_id(1)
    nj = pl.num_programs(1)

    part = jnp.sum(x_ref[0], axis=0, keepdims=True)

    @pl.when(j == 0)
    def _init():
        acc_ref[pl.ds(b, 1), :] = part

    @pl.when(j > 0)
    def _accum():
        acc_ref[pl.ds(b, 1), :] += part

    @pl.when((b == B - 1) & (j == nj - 1))
    def _finish():
        pooled = acc_ref[...] * (1.0 / S)
        h = lax.dot_general(
            pooled, w1_ref[...], (((1,), (0,)), ((), ())),
            preferred_element_type=jnp.float32,
        ) + b1_ref[...]
        h = 0.5 * h * (1.0 + lax.erf(h * (2.0 ** -0.5)))
        logits = lax.dot_general(
            h, w2_ref[...], (((1,), (0,)), ((), ())),
            preferred_element_type=jnp.float32,
        ) + b2_ref[...]
        m = jnp.max(logits, axis=-1, keepdims=True)
        e = jnp.exp(logits - m)
        out_ref[...] = e / jnp.sum(e, axis=-1, keepdims=True)


@jax.jit
def kernel(x, W1, b1, W2, b2):
    grid = (B, S // S_BLK)
    out = pl.pallas_call(
        _router_kernel,
        grid=grid,
        in_specs=[
            pl.BlockSpec((1, S_BLK, D), lambda b, j: (b, j, 0)),
            pl.BlockSpec((D, H), lambda b, j: (0, 0)),
            pl.BlockSpec((1, H), lambda b, j: (0, 0)),
            pl.BlockSpec((H, E), lambda b, j: (0, 0)),
            pl.BlockSpec((1, E), lambda b, j: (0, 0)),
        ],
        out_specs=pl.BlockSpec((B, E), lambda b, j: (0, 0)),
        out_shape=jax.ShapeDtypeStruct((B, E), jnp.float32),
        scratch_shapes=[pltpu.VMEM((B, D), jnp.float32)],
        compiler_params=pltpu.CompilerParams(
            dimension_semantics=("arbitrary", "arbitrary"),
        ),
    )(x, W1, b1.reshape(1, H), W2, b2.reshape(1, E))
    return out
